# baseline (speedup 1.0000x reference)
import jax
import jax.numpy as jnp
from jax.experimental import pallas as pl
from jax.experimental.pallas import tpu as pltpu


def _pack_body(slot_ref, flag_ref, src_ref, fast_ref, slow_ref):
    del slot_ref
    i = pl.program_id(0)
    fast_ref[...] = src_ref[...]

    @pl.when(flag_ref[i] == 1)
    def _():
        slow_ref[...] = src_ref[...]


def kernel(frames):
    C, T, H, W = frames.shape
    alpha = 4
    n = T // alpha
    idx = jnp.linspace(0.0, float(T - 1), n).astype(jnp.int32)

    t = jnp.arange(T, dtype=jnp.int32)
    sel = (t[:, None] >= idx[None, :]).sum(axis=1).astype(jnp.int32) - 1
    sel = jnp.maximum(sel, 0)
    flag = jnp.isin(t, idx).astype(jnp.int32)
    c = jnp.arange(C, dtype=jnp.int32)
    slow_slot = (c[:, None] * n + sel[None, :]).reshape(-1)
    write_flag = jnp.broadcast_to(flag[None, :], (C, T)).reshape(-1)

    flat = frames.reshape(C * T, H, W)
    fast_flat, slow_flat = pl.pallas_call(
        _pack_body,
        grid_spec=pltpu.PrefetchScalarGridSpec(
            num_scalar_prefetch=2,
            grid=(C * T,),
            in_specs=[pl.BlockSpec((1, H, W), lambda i, slot, flag: (i, 0, 0))],
            out_specs=[
                pl.BlockSpec((1, H, W), lambda i, slot, flag: (i, 0, 0)),
                pl.BlockSpec((1, H, W), lambda i, slot, flag: (slot[i], 0, 0)),
            ],
        ),
        out_shape=[
            jax.ShapeDtypeStruct((C * T, H, W), jnp.float32),
            jax.ShapeDtypeStruct((C * n, H, W), jnp.float32),
        ],
    )(slow_slot, write_flag, flat)
    return (slow_flat.reshape(C, n, H, W), fast_flat.reshape(C, T, H, W))
